# initial kernel scaffold (unmeasured)
import jax
import jax.numpy as jnp
from jax import lax
from jax.experimental import pallas as pl
from jax.experimental.pallas import tpu as pltpu

N_DEV = 32
V_PER = 4096
T = 512
D = 512
STEPS = 5


def kernel(ids, E):
    my = lax.axis_index("i")
    local = ids - my * V_PER
    owned = (local >= 0) & (local < V_PER)
    partial = jnp.where(owned[:, None], E[jnp.clip(local, 0, V_PER - 1)], 0.0)

    def body(x_ref, o_ref, recv_ref, send_sems, recv_sems, ack_sem):
        me = lax.axis_index("i")
        o_ref[...] = x_ref[...]
        for k in range(STEPS):
            partner = me ^ (1 << k)
            rdma = pltpu.make_async_remote_copy(
                src_ref=o_ref,
                dst_ref=recv_ref.at[k],
                send_sem=send_sems.at[k],
                recv_sem=recv_sems.at[k],
                device_id=(partner,),
                device_id_type=pl.DeviceIdType.MESH,
            )
            rdma.start()
            rdma.wait_send()
            rdma.wait_recv()
            o_ref[...] = o_ref[...] + recv_ref[k]
            pl.semaphore_signal(
                ack_sem, inc=1,
                device_id=(partner,), device_id_type=pl.DeviceIdType.MESH,
            )
            pl.semaphore_wait(ack_sem, 1)

    return pl.pallas_call(
        body,
        out_shape=jax.ShapeDtypeStruct((T, D), jnp.float32),
        in_specs=[pl.BlockSpec(memory_space=pltpu.VMEM)],
        out_specs=pl.BlockSpec(memory_space=pltpu.VMEM),
        scratch_shapes=[
            pltpu.VMEM((STEPS, T, D), jnp.float32),
            pltpu.SemaphoreType.DMA((STEPS,)),
            pltpu.SemaphoreType.DMA((STEPS,)),
            pltpu.SemaphoreType.REGULAR,
        ],
        compiler_params=pltpu.CompilerParams(collective_id=0),
    )(partial)


# baseline (device time: 108077 ns/iter reference)
import jax
import jax.numpy as jnp
from jax import lax
from jax.experimental import pallas as pl
from jax.experimental.pallas import tpu as pltpu

N_DEV = 32
V_PER = 4096
T = 512
D = 512
STEPS = 5


def kernel(ids, E):
    my = lax.axis_index("i")
    local = ids - my * V_PER
    owned = (local >= 0) & (local < V_PER)
    partial = jnp.where(owned[:, None], E[jnp.clip(local, 0, V_PER - 1)], 0.0)

    def body(x_ref, o_ref, recv_ref, send_sems, recv_sems, ack_sem):
        me = lax.axis_index("i")
        o_ref[...] = x_ref[...]
        for k in range(STEPS):
            partner = me ^ (1 << k)
            rdma = pltpu.make_async_remote_copy(
                src_ref=o_ref,
                dst_ref=recv_ref.at[k],
                send_sem=send_sems.at[k],
                recv_sem=recv_sems.at[k],
                device_id=(partner,),
                device_id_type=pl.DeviceIdType.MESH,
            )
            rdma.start()
            rdma.wait_send()
            rdma.wait_recv()
            o_ref[...] = o_ref[...] + recv_ref[k]
            pl.semaphore_signal(
                ack_sem, inc=1,
                device_id=(partner,), device_id_type=pl.DeviceIdType.MESH,
            )
            pl.semaphore_wait(ack_sem, 1)

    return pl.pallas_call(
        body,
        out_shape=jax.ShapeDtypeStruct((T, D), jnp.float32),
        in_specs=[pl.BlockSpec(memory_space=pltpu.VMEM)],
        out_specs=pl.BlockSpec(memory_space=pltpu.VMEM),
        scratch_shapes=[
            pltpu.VMEM((STEPS, T, D), jnp.float32),
            pltpu.SemaphoreType.DMA((STEPS,)),
            pltpu.SemaphoreType.DMA((STEPS,)),
            pltpu.SemaphoreType.REGULAR,
        ],
    )(partial)


# device time: 67393 ns/iter; 1.6037x vs baseline; 1.6037x over previous
import jax
import jax.numpy as jnp
from jax import lax
from jax.experimental import pallas as pl
from jax.experimental.pallas import tpu as pltpu

N_DEV = 32
V_PER = 4096
T = 512
D = 512
STEPS = 5

_RS_OFF = [0, 256, 384, 448, 480]


def kernel(ids, E):
    my = lax.axis_index("i")
    local = ids - my * V_PER
    owned = (local >= 0) & (local < V_PER)
    partial = jnp.where(owned[:, None], E[jnp.clip(local, 0, V_PER - 1)], 0.0)

    def body(x_ref, o_ref, recv_ref, rs_send, rs_recv, ag_send, ag_recv):
        me = lax.axis_index("i")
        o_ref[...] = x_ref[...]

        lo = jnp.int32(0)
        sz = T
        for s in range(STEPS):
            b = STEPS - 1 - s
            half = sz // 2
            bit = (me >> b) & 1
            keep_lo = pl.multiple_of(lo + bit * half, 16)
            send_lo = pl.multiple_of(lo + (1 - bit) * half, 16)
            partner = me ^ (1 << b)
            rdma = pltpu.make_async_remote_copy(
                src_ref=o_ref.at[pl.ds(send_lo, half)],
                dst_ref=recv_ref.at[pl.ds(_RS_OFF[s], half)],
                send_sem=rs_send.at[s],
                recv_sem=rs_recv.at[s],
                device_id=(partner,),
                device_id_type=pl.DeviceIdType.MESH,
            )
            rdma.start()
            rdma.wait_send()
            rdma.wait_recv()
            o_ref[pl.ds(keep_lo, half)] = (
                o_ref[pl.ds(keep_lo, half)]
                + recv_ref[pl.ds(_RS_OFF[s], half)]
            )
            lo = keep_lo
            sz = half

        for s in range(STEPS):
            partner = me ^ (1 << s)
            lo = pl.multiple_of(lo, 16)
            rdma = pltpu.make_async_remote_copy(
                src_ref=o_ref.at[pl.ds(lo, sz)],
                dst_ref=o_ref.at[pl.ds(lo, sz)],
                send_sem=ag_send.at[s],
                recv_sem=ag_recv.at[s],
                device_id=(partner,),
                device_id_type=pl.DeviceIdType.MESH,
            )
            rdma.start()
            rdma.wait_send()
            rdma.wait_recv()
            lo = lo - (lo & sz)
            sz = sz * 2

    return pl.pallas_call(
        body,
        out_shape=jax.ShapeDtypeStruct((T, D), jnp.float32),
        in_specs=[pl.BlockSpec(memory_space=pltpu.VMEM)],
        out_specs=pl.BlockSpec(memory_space=pltpu.VMEM),
        scratch_shapes=[
            pltpu.VMEM((T, D), jnp.float32),
            pltpu.SemaphoreType.DMA((STEPS,)),
            pltpu.SemaphoreType.DMA((STEPS,)),
            pltpu.SemaphoreType.DMA((STEPS,)),
            pltpu.SemaphoreType.DMA((STEPS,)),
        ],
    )(partial)
